# baseline (speedup 1.0000x reference)
import functools

import jax
import jax.numpy as jnp
from jax.experimental import pallas as pl
from jax.experimental.pallas import tpu as pltpu

_NEG = -1e30


def _encoder_kernel(seq_ref, wenc_ref, benc_ref, wproj_ref, bproj_ref,
                    out_ref, acc_ref):
    i = pl.program_id(1)
    x = seq_ref[0]
    h = jnp.dot(x, wenc_ref[...], preferred_element_type=jnp.float32)
    h = jax.nn.gelu(h + benc_ref[...])
    e = jnp.dot(h, wproj_ref[...], preferred_element_type=jnp.float32)
    e = e + bproj_ref[...]

    bt = e.shape[0]
    row = jax.lax.broadcasted_iota(jnp.int32, (bt, bt), 0)
    col = jax.lax.broadcasted_iota(jnp.int32, (bt, bt), 1)
    tril = (row >= col).astype(jnp.float32)
    cb = jnp.dot(tril, e, preferred_element_type=jnp.float32)

    @pl.when(i == 0)
    def _():
        acc_ref[...] = jnp.zeros_like(acc_ref)

    out = cb + acc_ref[...]
    out_ref[0] = out
    acc_ref[...] = out[bt - 1:bt, :]


def _dp_kernel(cumt_ref, durrev_ref, coeffs_ref, startl_ref, len_ref,
               out_ref, *, T, K, C, B):
    L = C * B
    dur_r = [durrev_ref[r] for r in range(C)]
    cm_r = [coeffs_ref[r] for r in range(C)]
    lenv = len_ref[...]

    def row_lse(ts):
        mx = jnp.max(ts, axis=0, keepdims=True)
        return mx + jnp.log(jnp.sum(jnp.exp(ts - mx), axis=0, keepdims=True))

    def body(t, carry):
        Gs, pacc = carry
        curs = [cumt_ref[r, pl.ds(t + K, 1), :] for r in range(C)]
        scores = [Gs[r] + dur_r[r] for r in range(C)]
        alpha0 = curs[0] + row_lse(scores[0])
        pacc = jnp.where(t == lenv, alpha0, pacc)
        new_Gs = []
        for r in range(C):
            ts = jnp.concatenate(
                [scores[rp] + (curs[rp] + cm_r[r][rp:rp + 1, :])
                 for rp in range(C)], axis=0)
            m_r = row_lse(ts)
            new_Gs.append(
                jnp.concatenate([Gs[r][1:], m_r - curs[r]], axis=0))
        return (tuple(new_Gs), pacc)

    negk = jnp.full((K - 1, L), _NEG, jnp.float32)
    G0 = tuple(jnp.concatenate([negk, startl_ref[r:r + 1, :]], axis=0)
               for r in range(C))
    p0 = jnp.full((1, L), _NEG, jnp.float32)
    _, pacc = jax.lax.fori_loop(1, T + 1, body, (G0, p0), unroll=2)
    p2 = jnp.concatenate([pacc, pacc], axis=1)
    ts = jnp.concatenate(
        [p2[:, (C - r) * B:(C - r) * B + L] for r in range(C)], axis=0)
    out_ref[...] = row_lse(ts)


def kernel(sequence, lengths, W_enc, b_enc, W_proj, b_proj, trans, start, dur):
    B, T, D = sequence.shape
    HID = W_enc.shape[1]
    C = W_proj.shape[1]
    K = dur.shape[1]
    L = C * B
    BT = 512
    NT = T // BT

    cum_body = pl.pallas_call(
        _encoder_kernel,
        grid=(B, NT),
        in_specs=[
            pl.BlockSpec((1, BT, D), lambda b, i: (b, i, 0)),
            pl.BlockSpec((D, HID), lambda b, i: (0, 0)),
            pl.BlockSpec((1, HID), lambda b, i: (0, 0)),
            pl.BlockSpec((HID, C), lambda b, i: (0, 0)),
            pl.BlockSpec((1, C), lambda b, i: (0, 0)),
        ],
        out_specs=pl.BlockSpec((1, BT, C), lambda b, i: (b, i, 0)),
        out_shape=jax.ShapeDtypeStruct((B, T, C), jnp.float32),
        scratch_shapes=[pltpu.VMEM((1, C), jnp.float32)],
        compiler_params=pltpu.CompilerParams(
            dimension_semantics=("parallel", "arbitrary")),
    )(sequence, W_enc, b_enc.reshape(1, HID), W_proj, b_proj.reshape(1, C))

    cum = jnp.concatenate(
        [jnp.zeros((B, 1, C), cum_body.dtype), cum_body], axis=1)

    cidx = jnp.arange(C)
    rot = (cidx[None, :] - cidx[:, None]) % C
    cumt0 = cum.transpose(1, 2, 0)
    cumt = cumt0[:, rot, :].transpose(1, 0, 2, 3).reshape(C, T + 1, L)
    cumt = jnp.concatenate(
        [jnp.zeros((C, K, L), jnp.float32), cumt], axis=1)
    durrev0 = dur[:, ::-1].T
    durrev = jnp.repeat(durrev0[:, rot].transpose(1, 0, 2), B, axis=2)
    coeffs = jnp.repeat(
        trans[(cidx[None, None, :] - cidx[None, :, None]) % C,
              (cidx[None, None, :] - cidx[:, None, None]) % C],
        B, axis=2)
    startl = jnp.repeat(start[rot], B, axis=1)
    lenl = jnp.tile(lengths, C).reshape(1, L).astype(jnp.int32)

    part_l = pl.pallas_call(
        functools.partial(_dp_kernel, T=T, K=K, C=C, B=B),
        out_shape=jax.ShapeDtypeStruct((1, L), jnp.float32),
        compiler_params=pltpu.CompilerParams(
            vmem_limit_bytes=50 * 1024 * 1024),
    )(cumt, durrev, coeffs, startl, lenl)

    partition = part_l[0, 0:B]
    return partition, cum


# speedup vs baseline: 1.1640x; 1.1640x over previous
import functools

import jax
import jax.numpy as jnp
from jax.experimental import pallas as pl
from jax.experimental.pallas import tpu as pltpu

_NEG = -1e30


def _encoder_kernel(seq_ref, wenc_ref, benc_ref, wproj_ref, bproj_ref,
                    out_ref, acc_ref):
    i = pl.program_id(1)
    x = seq_ref[0]
    h = jnp.dot(x, wenc_ref[...], preferred_element_type=jnp.float32)
    h = jax.nn.gelu(h + benc_ref[...])
    e = jnp.dot(h, wproj_ref[...], preferred_element_type=jnp.float32)
    e = e + bproj_ref[...]

    bt = e.shape[0]
    row = jax.lax.broadcasted_iota(jnp.int32, (bt, bt), 0)
    col = jax.lax.broadcasted_iota(jnp.int32, (bt, bt), 1)
    tril = (row >= col).astype(jnp.float32)
    cb = jnp.dot(tril, e, preferred_element_type=jnp.float32)

    @pl.when(i == 0)
    def _():
        acc_ref[...] = jnp.zeros_like(acc_ref)

    out = cb + acc_ref[...]
    out_ref[0] = out
    acc_ref[...] = out[bt - 1:bt, :]


def _dp_kernel(cumt_ref, durrev_ref, coeffs_ref, startl_ref, len_ref,
               out_ref, *, T, K, C, B):
    L = C * B
    dur_r = [durrev_ref[r] for r in range(C)]
    cm_r = [coeffs_ref[r] for r in range(C)]
    lenv = len_ref[...]

    def row_lse(ts):
        mx = jnp.max(ts, axis=0, keepdims=True)
        return mx + jnp.log(jnp.sum(jnp.exp(ts - mx), axis=0, keepdims=True))

    def body(t, carry):
        Gs, paccs = carry
        alphas, curs = [], []
        for r in range(C):
            cur = cumt_ref[r, pl.ds(t + K, 1), :]
            alphas.append(cur + row_lse(Gs[r] + dur_r[r]))
            curs.append(cur)
        new_Gs, new_paccs = [], []
        for r in range(C):
            ts = jnp.concatenate(
                [alphas[rp] + cm_r[r][rp:rp + 1, :] for rp in range(C)], axis=0)
            m_r = row_lse(ts)
            new_Gs.append(
                jnp.concatenate([Gs[r][1:], m_r - curs[r]], axis=0))
            new_paccs.append(jnp.where(t == lenv, alphas[r], paccs[r]))
        return (tuple(new_Gs), tuple(new_paccs))

    negk = jnp.full((K - 1, L), _NEG, jnp.float32)
    G0 = tuple(jnp.concatenate([negk, startl_ref[r:r + 1, :]], axis=0)
               for r in range(C))
    p0 = tuple(jnp.full((1, L), _NEG, jnp.float32) for _ in range(C))
    _, paccs = jax.lax.fori_loop(1, T + 1, body, (G0, p0), unroll=4)
    out_ref[...] = row_lse(jnp.concatenate(list(paccs), axis=0))


def kernel(sequence, lengths, W_enc, b_enc, W_proj, b_proj, trans, start, dur):
    B, T, D = sequence.shape
    HID = W_enc.shape[1]
    C = W_proj.shape[1]
    K = dur.shape[1]
    L = C * B
    BT = 512
    NT = T // BT

    cum_body = pl.pallas_call(
        _encoder_kernel,
        grid=(B, NT),
        in_specs=[
            pl.BlockSpec((1, BT, D), lambda b, i: (b, i, 0)),
            pl.BlockSpec((D, HID), lambda b, i: (0, 0)),
            pl.BlockSpec((1, HID), lambda b, i: (0, 0)),
            pl.BlockSpec((HID, C), lambda b, i: (0, 0)),
            pl.BlockSpec((1, C), lambda b, i: (0, 0)),
        ],
        out_specs=pl.BlockSpec((1, BT, C), lambda b, i: (b, i, 0)),
        out_shape=jax.ShapeDtypeStruct((B, T, C), jnp.float32),
        scratch_shapes=[pltpu.VMEM((1, C), jnp.float32)],
        compiler_params=pltpu.CompilerParams(
            dimension_semantics=("parallel", "arbitrary")),
    )(sequence, W_enc, b_enc.reshape(1, HID), W_proj, b_proj.reshape(1, C))

    cum = jnp.concatenate(
        [jnp.zeros((B, 1, C), cum_body.dtype), cum_body], axis=1)

    cidx = jnp.arange(C)
    rot = (cidx[None, :] - cidx[:, None]) % C
    cumt0 = cum.transpose(1, 2, 0)
    cumt = cumt0[:, rot, :].transpose(1, 0, 2, 3).reshape(C, T + 1, L)
    cumt = jnp.concatenate(
        [jnp.zeros((C, K, L), jnp.float32), cumt], axis=1)
    durrev0 = dur[:, ::-1].T
    durrev = jnp.repeat(durrev0[:, rot].transpose(1, 0, 2), B, axis=2)
    coeffs = jnp.repeat(
        trans[(cidx[None, None, :] - cidx[None, :, None]) % C,
              (cidx[None, None, :] - cidx[:, None, None]) % C],
        B, axis=2)
    startl = jnp.repeat(start[rot], B, axis=1)
    lenl = jnp.tile(lengths, C).reshape(1, L).astype(jnp.int32)

    part_l = pl.pallas_call(
        functools.partial(_dp_kernel, T=T, K=K, C=C, B=B),
        out_shape=jax.ShapeDtypeStruct((1, L), jnp.float32),
        compiler_params=pltpu.CompilerParams(
            vmem_limit_bytes=50 * 1024 * 1024),
    )(cumt, durrev, coeffs, startl, lenl)

    partition = part_l[0, 0:B]
    return partition, cum
